# Initial kernel scaffold; baseline (speedup 1.0000x reference)
import jax
import jax.numpy as jnp
from jax.experimental import pallas as pl


def kernel(xyz, center):
    raise NotImplementedError("write your pallas kernel here")


# speedup vs baseline: 6.2453x; 6.2453x over previous
import numpy as np

import jax
import jax.numpy as jnp
from jax import lax
from jax.experimental import pallas as pl
from jax.experimental.pallas import tpu as pltpu
from jax.experimental.pallas import tpu_sc as plsc

B = 4
NPOINT = 1024
N = 16384
K = 32
L = 16
NV = N // L
NSEG = 32
SEGV = NV // NSEG
NTEC = 32
CPT = (B * NPOINT) // NTEC
TPB = NTEC // B
CAP = N + 2 * L

F32_INF = np.float32(np.inf)
F32_NINF = np.float32(-np.inf)
I32_MAX = np.int32(2**31 - 1)


def _knn_body(xt, ct, out, xv, yv, zv, cxv, cyv, czv, distv, cand_d, cand_i,
              outbuf):
    wid = lax.axis_index("s") * 2 + lax.axis_index("c")
    b = wid // TPB
    c0 = (wid % TPB) * CPT

    pltpu.sync_copy(xt.at[pl.ds((b * 3 + 0) * N, N)], xv)
    pltpu.sync_copy(xt.at[pl.ds((b * 3 + 1) * N, N)], yv)
    pltpu.sync_copy(xt.at[pl.ds((b * 3 + 2) * N, N)], zv)
    pltpu.sync_copy(ct.at[pl.ds(((b * 3 + 0) * NPOINT + c0) * L, CPT * L)], cxv)
    pltpu.sync_copy(ct.at[pl.ds(((b * 3 + 1) * NPOINT + c0) * L, CPT * L)], cyv)
    pltpu.sync_copy(ct.at[pl.ds(((b * 3 + 2) * NPOINT + c0) * L, CPT * L)], czv)

    iota = lax.iota(jnp.int32, L)

    def center_body(ci, _):
        cx = cxv[pl.ds(ci * L, L)]
        cy = cyv[pl.ds(ci * L, L)]
        cz = czv[pl.ds(ci * L, L)]

        def seg_body(s, t):
            def vreg_body(j, m):
                off = (s * SEGV + j) * L
                x = xv[pl.ds(off, L)]
                y = yv[pl.ds(off, L)]
                z = zv[pl.ds(off, L)]
                dx = cx - x
                dy = cy - y
                dz = cz - z
                d = (dx * dx + dy * dy) + dz * dz
                distv[pl.ds(off, L)] = d
                return jnp.minimum(m, d)

            m = lax.fori_loop(0, SEGV, vreg_body,
                              jnp.full((L,), F32_INF, jnp.float32))
            return jnp.maximum(t, jnp.min(m))

        t = lax.fori_loop(0, NSEG, seg_body, jnp.float32(F32_NINF))

        def p2_body(i, off):
            o16 = i * L
            d = distv[pl.ds(o16, L)]
            sel = d <= t
            pc = plsc.cumsum(sel.astype(jnp.int32))
            dest = off + pc - 1
            plsc.store_scatter(cand_d, [dest], d, mask=sel)
            plsc.store_scatter(cand_i, [dest], iota + o16, mask=sel)
            return off + jnp.max(pc)

        c = lax.fori_loop(0, NV, p2_body, jnp.int32(0))
        nv = (c + L - 1) // L

        def ext_body(j, state):
            pd, pi, ov0, ov1 = state

            def scan_body(v, bs):
                bd, bi = bs
                o16 = v * L
                d = cand_d[pl.ds(o16, L)]
                ii = cand_i[pl.ds(o16, L)]
                ok = ((iota + o16) < c) & ((d > pd) | ((d == pd) & (ii > pi)))
                d = jnp.where(ok, d, F32_INF)
                ii = jnp.where(ok, ii, I32_MAX)
                better = (d < bd) | ((d == bd) & (ii < bi))
                return (jnp.where(better, d, bd), jnp.where(better, ii, bi))

            bd, bi = lax.fori_loop(0, nv, scan_body,
                                   (jnp.full((L,), F32_INF, jnp.float32),
                                    jnp.full((L,), I32_MAX, jnp.int32)))
            dmin = jnp.min(bd)
            imin = jnp.min(jnp.where(bd == dmin, bi, I32_MAX))
            ov0 = jnp.where((j < L) & (iota == j), imin, ov0)
            ov1 = jnp.where((j >= L) & (iota == j - L), imin, ov1)
            return (dmin, imin, ov0, ov1)

        zero16 = jnp.zeros((L,), jnp.int32)
        _, _, ov0, ov1 = lax.fori_loop(
            0, K, ext_body,
            (jnp.float32(F32_NINF), jnp.int32(-1), zero16, zero16))
        outbuf[pl.ds(ci * K, L)] = ov0
        outbuf[pl.ds(ci * K + L, L)] = ov1
        return 0

    lax.fori_loop(0, CPT, center_body, 0)
    pltpu.sync_copy(outbuf, out.at[pl.ds(wid * (CPT * K), CPT * K)])


@jax.jit
def _knn(xt, ct):
    f = pl.kernel(
        _knn_body,
        out_type=jax.ShapeDtypeStruct((B * NPOINT * K,), jnp.int32),
        mesh=plsc.VectorSubcoreMesh(core_axis_name="c", subcore_axis_name="s"),
        compiler_params=pltpu.CompilerParams(needs_layout_passes=False),
        scratch_types=[
            pltpu.VMEM((N,), jnp.float32),
            pltpu.VMEM((N,), jnp.float32),
            pltpu.VMEM((N,), jnp.float32),
            pltpu.VMEM((CPT * L,), jnp.float32),
            pltpu.VMEM((CPT * L,), jnp.float32),
            pltpu.VMEM((CPT * L,), jnp.float32),
            pltpu.VMEM((N,), jnp.float32),
            pltpu.VMEM((CAP,), jnp.float32),
            pltpu.VMEM((CAP,), jnp.int32),
            pltpu.VMEM((CPT * K,), jnp.int32),
        ],
    )
    return f(xt, ct)


def kernel(xyz, center):
    xt = jnp.transpose(xyz, (0, 2, 1)).reshape(B * 3 * N)
    ct = jnp.repeat(jnp.transpose(center, (0, 2, 1)).reshape(B * 3 * NPOINT), L)
    return _knn(xt, ct).reshape(B, NPOINT, K)
